# baseline (speedup 1.0000x reference)
import functools

import jax
import jax.numpy as jnp
import numpy as np
from jax import lax
from jax.experimental import pallas as pl
from jax.experimental.pallas import tpu as pltpu
from jax.experimental.pallas import tpu_sc as plsc

NUM_GT = 20000
P = 128
NBINS = 50
NC, NS, L = 2, 16, 16
NW = NC * NS
NGROUPS = NUM_GT // L
GPW = NGROUPS // NW
GREM = NGROUPS % NW
GMAX = GPW + 1
GT_PAD = NUM_GT + GMAX * L

_f32 = jnp.float32
_i32 = jnp.int32


def _sc_body(pb_ref, gt_ref, hist_out, sum_out, boxbuf, gtbuf, histbuf, sumbuf):
    cid = lax.axis_index("c")
    sid = lax.axis_index("s")
    wid = cid * NS + sid
    ngroups = GPW + (wid < GREM).astype(_i32)
    gstart = wid * GPW + jnp.minimum(wid, GREM)

    idx16 = jnp.arange(L, dtype=_i32)
    c0 = jnp.zeros((L,), _i32)
    c1 = c0 + 1
    c2 = c0 + 2
    c3 = c0 + 3

    pltpu.sync_copy(gt_ref.at[pl.ds(gstart * L, GMAX * L)], gtbuf)

    for b in range(NBINS):
        histbuf[b] = jnp.zeros((L,), _i32)

    rows0 = idx16 * P

    def group_step(i, sumvec):
        valid = (i < ngroups).astype(_i32)
        gi = jnp.minimum(i, ngroups - 1)
        g = gstart + gi
        pltpu.sync_copy(pb_ref.at[pl.ds(g * (L * P), L * P)], boxbuf)

        grow = gi * L + idx16
        gx1 = plsc.load_gather(gtbuf, [grow, c0])
        gy1 = plsc.load_gather(gtbuf, [grow, c1])
        gx2 = plsc.load_gather(gtbuf, [grow, c2])
        gy2 = plsc.load_gather(gtbuf, [grow, c3])
        area2 = (gx2 - gx1) * (gy2 - gy1)

        def box_step(j, carry):
            m, rows = carry
            x1 = plsc.load_gather(boxbuf, [rows, c0])
            y1 = plsc.load_gather(boxbuf, [rows, c1])
            x2 = plsc.load_gather(boxbuf, [rows, c2])
            y2 = plsc.load_gather(boxbuf, [rows, c3])
            a1 = (x2 - x1) * (y2 - y1)
            ltx = jnp.maximum(x1, gx1)
            lty = jnp.maximum(y1, gy1)
            rbx = jnp.minimum(x2, gx2)
            rby = jnp.minimum(y2, gy2)
            wx = jnp.maximum(rbx - ltx, _f32(0.0))
            wy = jnp.maximum(rby - lty, _f32(0.0))
            ov = wx * wy
            un = jnp.maximum(a1 + area2 - ov, _f32(1e-6))
            iou = ov / un
            return jnp.maximum(m, iou), rows + 1

        m, _ = lax.fori_loop(0, P, box_step, (jnp.zeros((L,), _f32), rows0))

        bins = (m / _f32(0.02)).astype(_i32)
        bins = jnp.minimum(jnp.maximum(bins, 0), NBINS - 1)
        plsc.addupdate_scatter(histbuf, [bins, idx16], c0 + valid)
        return sumvec + m * valid.astype(_f32)

    sumvec = lax.fori_loop(0, GMAX, group_step, jnp.zeros((L,), _f32))

    sumbuf[...] = sumvec
    pltpu.sync_copy(histbuf, hist_out.at[wid])
    pltpu.sync_copy(sumbuf, sum_out.at[wid])


@functools.partial(pl.kernel,
                   out_type=(jax.ShapeDtypeStruct((NW, NBINS, L), _i32),
                             jax.ShapeDtypeStruct((NW, L), _f32)),
                   mesh=plsc.VectorSubcoreMesh(core_axis_name="c",
                                               subcore_axis_name="s",
                                               num_cores=NC, num_subcores=NS),
                   scratch_types=[pltpu.VMEM((L * P, 4), _f32),
                                  pltpu.VMEM((GMAX * L, 4), _f32),
                                  pltpu.VMEM((NBINS, L), _i32),
                                  pltpu.VMEM((L,), _f32)],
                   compiler_params=pltpu.CompilerParams(
                       needs_layout_passes=False,
                       use_tc_tiling_on_sc=False))
def _sc_iou_hist(pb_ref, gt_ref, hist_out, sum_out, boxbuf, gtbuf, histbuf, sumbuf):
    _sc_body(pb_ref, gt_ref, hist_out, sum_out, boxbuf, gtbuf, histbuf, sumbuf)


@jax.jit
def kernel(pseudo_boxes, gt_bboxes):
    pb = pseudo_boxes.reshape(NUM_GT * P, 4)
    gt = gt_bboxes.reshape(NUM_GT, 4)
    gt = jnp.pad(gt, ((0, GT_PAD - NUM_GT), (0, 0)))
    hist, sums = _sc_iou_hist(pb, gt)
    iou_bin = jnp.sum(hist, axis=(0, 2), dtype=_i32)
    mean_iou = jnp.sum(sums) / NUM_GT
    return iou_bin, mean_iou


# speedup vs baseline: 1.1794x; 1.1794x over previous
import functools

import jax
import jax.numpy as jnp
import numpy as np
from jax import lax
from jax.experimental import pallas as pl
from jax.experimental.pallas import tpu as pltpu
from jax.experimental.pallas import tpu_sc as plsc

NUM_GT = 20000
P = 128
NBINS = 50
NC, NS, L = 2, 16, 16
NW = NC * NS
NGROUPS = NUM_GT // L
GPW = NGROUPS // NW
GREM = NGROUPS % NW
GMAX = GPW + 1
GT_PAD = NUM_GT + GMAX * L
GSZ = L * P
FLAT = GSZ * 4
U = 8

_f32 = jnp.float32
_i32 = jnp.int32


def _sc_body(pb_ref, gt_ref, hist_out, sum_out, buf0, buf1, gtbuf, histbuf,
             sumbuf, sem0, sem1):
    cid = lax.axis_index("c")
    sid = lax.axis_index("s")
    wid = cid * NS + sid
    ngroups = GPW + (wid < GREM).astype(_i32)
    gstart = wid * GPW + jnp.minimum(wid, GREM)

    idx16 = jnp.arange(L, dtype=_i32)
    c0 = jnp.zeros((L,), _i32)

    pltpu.sync_copy(gt_ref.at[pl.ds(gstart * L, GMAX * L)], gtbuf)

    for b in range(NBINS):
        histbuf[b] = jnp.zeros((L,), _i32)

    def dma_start(lg, buf, sem):
        g = gstart + jnp.minimum(lg, ngroups - 1)
        pltpu.async_copy(pb_ref.at[pl.ds(g * FLAT, FLAT)], buf, sem)

    def dma_wait(buf, sem):
        pltpu.make_async_copy(pb_ref.at[pl.ds(0, FLAT)], buf, sem).wait()

    ix0 = idx16 * (P * 4)

    def compute_group(fbuf, lg, sumvec):
        valid = (lg < ngroups).astype(_i32)
        grow = jnp.minimum(lg, ngroups - 1) * L + idx16
        gx1 = plsc.load_gather(gtbuf, [grow, c0])
        gy1 = plsc.load_gather(gtbuf, [grow, c0 + 1])
        gx2 = plsc.load_gather(gtbuf, [grow, c0 + 2])
        gy2 = plsc.load_gather(gtbuf, [grow, c0 + 3])
        area2 = (gx2 - gx1) * (gy2 - gy1)

        def box_step(t, carry):
            ovb, unb, ix = carry
            for u in range(U):
                ia = ix + (4 * u)
                x1 = plsc.load_gather(fbuf, [ia])
                y1 = plsc.load_gather(fbuf, [ia + 1])
                x2 = plsc.load_gather(fbuf, [ia + 2])
                y2 = plsc.load_gather(fbuf, [ia + 3])
                a1 = (x2 - x1) * (y2 - y1)
                ltx = jnp.maximum(x1, gx1)
                lty = jnp.maximum(y1, gy1)
                rbx = jnp.minimum(x2, gx2)
                rby = jnp.minimum(y2, gy2)
                wx = jnp.maximum(rbx - ltx, _f32(0.0))
                wy = jnp.maximum(rby - lty, _f32(0.0))
                ov = wx * wy
                un = jnp.maximum(a1 + area2 - ov, _f32(1e-6))
                better = ov * unb > ovb * un
                ovb = jnp.where(better, ov, ovb)
                unb = jnp.where(better, un, unb)
            return ovb, unb, ix + 4 * U

        ovb, unb, _ = lax.fori_loop(
            0, P // U, box_step,
            (jnp.zeros((L,), _f32), jnp.ones((L,), _f32), ix0))

        m = ovb / unb
        bins = (m / _f32(0.02)).astype(_i32)
        bins = jnp.minimum(jnp.maximum(bins, 0), NBINS - 1)
        plsc.addupdate_scatter(histbuf, [bins, idx16], c0 + valid)
        return sumvec + m * valid.astype(_f32)

    dma_start(0, buf0, sem0)
    dma_start(1, buf1, sem1)

    def pair_step(i2, sumvec):
        la = i2 * 2
        dma_wait(buf0, sem0)
        sumvec = compute_group(buf0, la, sumvec)
        dma_start(la + 2, buf0, sem0)
        dma_wait(buf1, sem1)
        sumvec = compute_group(buf1, la + 1, sumvec)
        dma_start(la + 3, buf1, sem1)
        return sumvec

    sumvec = lax.fori_loop(0, GMAX // 2 - 1, pair_step, jnp.zeros((L,), _f32))

    dma_wait(buf0, sem0)
    sumvec = compute_group(buf0, GMAX - 2, sumvec)
    dma_wait(buf1, sem1)
    sumvec = compute_group(buf1, GMAX - 1, sumvec)

    sumbuf[...] = sumvec
    pltpu.sync_copy(histbuf, hist_out.at[wid])
    pltpu.sync_copy(sumbuf, sum_out.at[wid])


@functools.partial(pl.kernel,
                   out_type=(jax.ShapeDtypeStruct((NW, NBINS, L), _i32),
                             jax.ShapeDtypeStruct((NW, L), _f32)),
                   mesh=plsc.VectorSubcoreMesh(core_axis_name="c",
                                               subcore_axis_name="s",
                                               num_cores=NC, num_subcores=NS),
                   scratch_types=[pltpu.VMEM((FLAT,), _f32),
                                  pltpu.VMEM((FLAT,), _f32),
                                  pltpu.VMEM((GMAX * L, 4), _f32),
                                  pltpu.VMEM((NBINS, L), _i32),
                                  pltpu.VMEM((L,), _f32),
                                  pltpu.SemaphoreType.DMA,
                                  pltpu.SemaphoreType.DMA],
                   compiler_params=pltpu.CompilerParams(
                       needs_layout_passes=False,
                       use_tc_tiling_on_sc=False))
def _sc_iou_hist(pb_ref, gt_ref, hist_out, sum_out, buf0, buf1, gtbuf,
                 histbuf, sumbuf, sem0, sem1):
    _sc_body(pb_ref, gt_ref, hist_out, sum_out, buf0, buf1, gtbuf, histbuf,
             sumbuf, sem0, sem1)


@jax.jit
def kernel(pseudo_boxes, gt_bboxes):
    pb = pseudo_boxes.reshape(NUM_GT * P * 4)
    gt = gt_bboxes.reshape(NUM_GT, 4)
    gt = jnp.pad(gt, ((0, GT_PAD - NUM_GT), (0, 0)))
    hist, sums = _sc_iou_hist(pb, gt)
    iou_bin = jnp.sum(hist, axis=(0, 2), dtype=_i32)
    mean_iou = jnp.sum(sums) / NUM_GT
    return iou_bin, mean_iou
